# Initial kernel scaffold; baseline (speedup 1.0000x reference)
import jax
import jax.numpy as jnp
from jax.experimental import pallas as pl


def kernel(inputs_embeds, pos_table, ln_gamma, ln_beta):
    raise NotImplementedError("write your pallas kernel here")


# speedup vs baseline: 2.0593x; 2.0593x over previous
import functools

import jax
import jax.numpy as jnp
from jax.experimental import pallas as pl
from jax.experimental.pallas import tpu as pltpu

SEQ_LEN = 8192
D_MODEL = 768
BATCH = 4
EPS = 1e-12

BLOCK_ROWS = 512


def _ln_kernel(x_ref, pos_ref, gamma_ref, beta_ref, out_ref):
    x = x_ref[0] + pos_ref[...]
    mean = jnp.mean(x, axis=-1, keepdims=True)
    centered = x - mean
    var = jnp.mean(centered * centered, axis=-1, keepdims=True)
    normed = centered * jax.lax.rsqrt(var + EPS)
    out_ref[0] = normed * gamma_ref[...] + beta_ref[...]


@jax.jit
def kernel(inputs_embeds, pos_table, ln_gamma, ln_beta):
    num_seq_blocks = SEQ_LEN // BLOCK_ROWS
    grid = (num_seq_blocks, BATCH)
    return pl.pallas_call(
        _ln_kernel,
        grid=grid,
        in_specs=[
            pl.BlockSpec((1, BLOCK_ROWS, D_MODEL), lambda i, j: (j, i, 0)),
            pl.BlockSpec((BLOCK_ROWS, D_MODEL), lambda i, j: (i, 0)),
            pl.BlockSpec((D_MODEL,), lambda i, j: (0,)),
            pl.BlockSpec((D_MODEL,), lambda i, j: (0,)),
        ],
        out_specs=pl.BlockSpec((1, BLOCK_ROWS, D_MODEL), lambda i, j: (j, i, 0)),
        out_shape=jax.ShapeDtypeStruct((BATCH, SEQ_LEN, D_MODEL), jnp.float32),
        compiler_params=pltpu.CompilerParams(
            dimension_semantics=("arbitrary", "arbitrary"),
        ),
    )(inputs_embeds, pos_table, ln_gamma, ln_beta)


# speedup vs baseline: 2.4358x; 1.1828x over previous
import functools

import jax
import jax.numpy as jnp
from jax.experimental import pallas as pl
from jax.experimental.pallas import tpu as pltpu

SEQ_LEN = 8192
D_MODEL = 768
BATCH = 4
EPS = 1e-12

BLOCK_ROWS = 1024


def _ln_kernel(x_ref, pos_ref, gamma_ref, beta_ref, out_ref):
    x = x_ref[0] + pos_ref[...]
    mean = jnp.mean(x, axis=-1, keepdims=True)
    centered = x - mean
    var = jnp.mean(centered * centered, axis=-1, keepdims=True)
    normed = centered * jax.lax.rsqrt(var + EPS)
    out_ref[0] = normed * gamma_ref[...] + beta_ref[...]


@jax.jit
def kernel(inputs_embeds, pos_table, ln_gamma, ln_beta):
    num_seq_blocks = SEQ_LEN // BLOCK_ROWS
    grid = (num_seq_blocks, BATCH)
    return pl.pallas_call(
        _ln_kernel,
        grid=grid,
        in_specs=[
            pl.BlockSpec((1, BLOCK_ROWS, D_MODEL), lambda i, j: (j, i, 0)),
            pl.BlockSpec((BLOCK_ROWS, D_MODEL), lambda i, j: (i, 0)),
            pl.BlockSpec((D_MODEL,), lambda i, j: (0,)),
            pl.BlockSpec((D_MODEL,), lambda i, j: (0,)),
        ],
        out_specs=pl.BlockSpec((1, BLOCK_ROWS, D_MODEL), lambda i, j: (j, i, 0)),
        out_shape=jax.ShapeDtypeStruct((BATCH, SEQ_LEN, D_MODEL), jnp.float32),
        compiler_params=pltpu.CompilerParams(
            dimension_semantics=("arbitrary", "arbitrary"),
        ),
    )(inputs_embeds, pos_table, ln_gamma, ln_beta)


# speedup vs baseline: 2.6348x; 1.0817x over previous
import functools

import jax
import jax.numpy as jnp
from jax.experimental import pallas as pl
from jax.experimental.pallas import tpu as pltpu

SEQ_LEN = 8192
D_MODEL = 768
BATCH = 4
EPS = 1e-12

BLOCK_ROWS = 2048


def _ln_kernel(x_ref, pos_ref, gamma_ref, beta_ref, out_ref):
    x = x_ref[0] + pos_ref[...]
    mean = jnp.mean(x, axis=-1, keepdims=True)
    centered = x - mean
    var = jnp.mean(centered * centered, axis=-1, keepdims=True)
    normed = centered * jax.lax.rsqrt(var + EPS)
    out_ref[0] = normed * gamma_ref[...] + beta_ref[...]


@jax.jit
def kernel(inputs_embeds, pos_table, ln_gamma, ln_beta):
    num_seq_blocks = SEQ_LEN // BLOCK_ROWS
    grid = (num_seq_blocks, BATCH)
    return pl.pallas_call(
        _ln_kernel,
        grid=grid,
        in_specs=[
            pl.BlockSpec((1, BLOCK_ROWS, D_MODEL), lambda i, j: (j, i, 0)),
            pl.BlockSpec((BLOCK_ROWS, D_MODEL), lambda i, j: (i, 0)),
            pl.BlockSpec((D_MODEL,), lambda i, j: (0,)),
            pl.BlockSpec((D_MODEL,), lambda i, j: (0,)),
        ],
        out_specs=pl.BlockSpec((1, BLOCK_ROWS, D_MODEL), lambda i, j: (j, i, 0)),
        out_shape=jax.ShapeDtypeStruct((BATCH, SEQ_LEN, D_MODEL), jnp.float32),
        compiler_params=pltpu.CompilerParams(
            dimension_semantics=("arbitrary", "arbitrary"),
        ),
    )(inputs_embeds, pos_table, ln_gamma, ln_beta)
